# baseline (speedup 1.0000x reference)
import functools

import jax
import jax.numpy as jnp
from jax import lax
from jax.experimental import pallas as pl
from jax.experimental.pallas import tpu as pltpu
from jax.experimental.pallas import tpu_sc as plsc

N = 2048
D = 1024
E = 8
K = 2
F = 2048
TM = 128
NB = N * K + E * TM
NT = NB // TM
LANES = 128
NW = 32


def _router_body(x_ref, gw_ref, probs_ref, ti_ref, tp_ref, aux_ref,
                 pos_ref, counts_ref):
    x = x_ref[...]
    gw = gw_ref[...]
    logits = jnp.dot(x, gw, preferred_element_type=jnp.float32)

    lane = lax.broadcasted_iota(jnp.int32, (N, LANES), 1)
    valid = lane < E
    neg = jnp.float32(-1e30)
    lm = jnp.where(valid, logits, neg)
    m = jnp.max(lm, axis=1, keepdims=True)
    ex = jnp.where(valid, jnp.exp(lm - m), 0.0)
    s = jnp.sum(ex, axis=1, keepdims=True)
    p = ex / s
    probs_ref[...] = p[:, :E]

    big = jnp.int32(999)
    v0 = jnp.max(p, axis=1, keepdims=True)
    i0 = jnp.min(jnp.where((p == v0) & valid, lane, big), axis=1,
                 keepdims=True)
    p1 = jnp.where(valid & (lane != i0), p, -1.0)
    v1 = jnp.max(p1, axis=1, keepdims=True)
    i1 = jnp.min(jnp.where(p1 == v1, lane, big), axis=1, keepdims=True)
    ti_ref[...] = jnp.concatenate([i0, i1], axis=1)
    s2 = v0 + v1
    tp_ref[...] = jnp.concatenate([v0 / s2, v1 / s2], axis=1)

    ohA = jnp.where(lane == i0, 1.0, 0.0)
    ohB = jnp.where(lane == i1, 1.0, 0.0)
    cnt = jnp.sum(ohA + ohB, axis=0, keepdims=True)
    sp = jnp.sum(p, axis=0, keepdims=True)
    aux_ref[...] = (E / (N * N)) * jnp.sum(cnt * sp, axis=1, keepdims=True)

    r = lax.broadcasted_iota(jnp.int32, (TM, TM), 0)
    c = lax.broadcasted_iota(jnp.int32, (TM, TM), 1)
    tstrict = (r > c).astype(jnp.float32)
    mlt = (r < c).astype(jnp.float32)

    oh = jnp.concatenate([ohA, ohB], axis=0)
    carry = jnp.zeros((1, LANES), jnp.float32)
    rank_blocks = []
    for b in range(2 * N // TM):
        blk = lax.slice(oh, (b * TM, 0), ((b + 1) * TM, LANES))
        rank_blocks.append(
            jnp.dot(tstrict, blk, preferred_element_type=jnp.float32) + carry)
        carry = carry + jnp.sum(blk, axis=0, keepdims=True)
    ranks = jnp.concatenate(rank_blocks, axis=0)
    counts = carry
    cpad = jnp.ceil(counts / TM) * TM
    aoff = jnp.dot(cpad, mlt, preferred_element_type=jnp.float32)
    posf = jnp.sum(oh * (ranks + aoff), axis=1, keepdims=True)
    pos_ref[...] = posf.astype(jnp.int32)
    counts_ref[...] = counts


_router = pl.pallas_call(
    _router_body,
    out_shape=(
        jax.ShapeDtypeStruct((N, E), jnp.float32),
        jax.ShapeDtypeStruct((N, K), jnp.int32),
        jax.ShapeDtypeStruct((N, K), jnp.float32),
        jax.ShapeDtypeStruct((1, 1), jnp.float32),
        jax.ShapeDtypeStruct((2 * N, 1), jnp.int32),
        jax.ShapeDtypeStruct((1, LANES), jnp.float32),
    ),
)


_RPW = NB // NW


def _dispatch_body(pos_hbm, x_hbm, xs_hbm, tsbuf, posbuf, myids, rowbuf,
                   ts_sh, sem):
    s = lax.axis_index("s")
    c = lax.axis_index("c")

    @pl.when(s == 0)
    def _build_sorted_ids():
        def zero_body(i, carry):
            tsbuf[pl.ds(i * 16, 16)] = jnp.zeros((16,), jnp.int32)
            return carry
        lax.fori_loop(0, NB // 16, zero_body, 0)
        pltpu.sync_copy(pos_hbm, posbuf)

        def scat_body(i, carry):
            idx = posbuf[pl.ds(i * 16, 16)]
            vals = (i * 16 + lax.iota(jnp.int32, 16)) & (N - 1)
            plsc.store_scatter(tsbuf, [idx], vals)
            return carry
        lax.fori_loop(0, 2 * N // 16, scat_body, 0)
        pltpu.sync_copy(tsbuf, ts_sh)

    plsc.subcore_barrier()
    wid = s * 2 + c
    base = wid * _RPW
    pltpu.sync_copy(ts_sh.at[pl.ds(base, _RPW)], myids)
    for ch in range(_RPW // 32):
        pltpu.async_copy(x_hbm.at[myids.at[pl.ds(ch * 32, 32)]],
                         rowbuf, sem).wait()
        pltpu.sync_copy(rowbuf, xs_hbm.at[pl.ds(base + ch * 32, 32)])


def _ffn_body(te_ref, xs_ref, w1_ref, w2_ref, y_ref):
    xb = xs_ref[...].astype(jnp.bfloat16)
    h = jnp.dot(xb, w1_ref[0], preferred_element_type=jnp.float32)
    h = 0.5 * h * (1.0 + lax.erf(h * 0.7071067811865476))
    y_ref[...] = jnp.dot(h.astype(jnp.bfloat16), w2_ref[0],
                         preferred_element_type=jnp.float32)


_ffn = pl.pallas_call(
    _ffn_body,
    grid_spec=pltpu.PrefetchScalarGridSpec(
        num_scalar_prefetch=1,
        grid=(NT,),
        in_specs=[
            pl.BlockSpec((TM, D), lambda t, te: (t, 0)),
            pl.BlockSpec((1, D, F), lambda t, te: (te[t], 0, 0)),
            pl.BlockSpec((1, F, D), lambda t, te: (te[t], 0, 0)),
        ],
        out_specs=pl.BlockSpec((TM, D), lambda t, te: (t, 0)),
    ),
    out_shape=jax.ShapeDtypeStruct((NB, D), jnp.float32),
    compiler_params=pltpu.CompilerParams(
        dimension_semantics=("arbitrary",)),
)


_TPW = N // NW


def _combine_body(y_hbm, pos_hbm, tpa_hbm, tpb_hbm, out_hbm, idxa, idxb,
                  tpa_v, tpb_v, bufa, bufb, obuf, sema, semb):
    s = lax.axis_index("s")
    c = lax.axis_index("c")
    wid = s * 2 + c
    base = wid * _TPW
    for ch in range(_TPW // 16):
        tb = base + ch * 16
        pltpu.sync_copy(pos_hbm.at[pl.ds(tb, 16)], idxa)
        pltpu.sync_copy(pos_hbm.at[pl.ds(N + tb, 16)], idxb)
        pltpu.sync_copy(tpa_hbm.at[pl.ds(tb, 16)], tpa_v)
        pltpu.sync_copy(tpb_hbm.at[pl.ds(tb, 16)], tpb_v)
        ca = pltpu.async_copy(y_hbm.at[idxa], bufa, sema)
        cb = pltpu.async_copy(y_hbm.at[idxb], bufb, semb)
        ca.wait()
        cb.wait()
        tpav = tpa_v[...]
        tpbv = tpb_v[...]
        for i in range(16):
            a = tpav[i]
            b = tpbv[i]

            def row_body(j, carry, i=i, a=a, b=b):
                sl = pl.ds(j * 16, 16)
                obuf[i, sl] = bufa[i, sl] * a + bufb[i, sl] * b
                return carry
            lax.fori_loop(0, D // 16, row_body, 0)
        pltpu.sync_copy(obuf, out_hbm.at[pl.ds(tb, 16)])


@functools.cache
def _sc_kernels():
    mesh = plsc.VectorSubcoreMesh(core_axis_name="c", subcore_axis_name="s")
    dispatch = pl.kernel(
        _dispatch_body,
        mesh=mesh,
        out_type=jax.ShapeDtypeStruct((NB, D), jnp.float32),
        scratch_types=[
            pltpu.VMEM((NB,), jnp.int32),
            pltpu.VMEM((2 * N,), jnp.int32),
            pltpu.VMEM((_RPW,), jnp.int32),
            pltpu.VMEM((32, D), jnp.float32),
            pltpu.VMEM_SHARED((NB,), jnp.int32),
            pltpu.SemaphoreType.DMA,
        ],
        compiler_params=pltpu.CompilerParams(needs_layout_passes=False),
    )
    combine = pl.kernel(
        _combine_body,
        mesh=mesh,
        out_type=jax.ShapeDtypeStruct((N, D), jnp.float32),
        scratch_types=[
            pltpu.VMEM((16,), jnp.int32),
            pltpu.VMEM((16,), jnp.int32),
            pltpu.VMEM((16,), jnp.float32),
            pltpu.VMEM((16,), jnp.float32),
            pltpu.VMEM((16, D), jnp.float32),
            pltpu.VMEM((16, D), jnp.float32),
            pltpu.VMEM((16, D), jnp.float32),
            pltpu.SemaphoreType.DMA,
            pltpu.SemaphoreType.DMA,
        ],
    )
    return dispatch, combine


def kernel(x, gate_w, w1, w2):
    Bb, Tt, Dd = x.shape
    x_flat = x.reshape(N, D)
    gwp = jnp.pad(gate_w, ((0, 0), (0, LANES - E)))
    probs, ti, tp, aux, pos, counts = _router(x_flat, gwp)

    counts8 = counts[0, :E]
    cpad = jnp.ceil(counts8 / TM) * TM
    aoff = jnp.concatenate(
        [jnp.zeros((1,), jnp.float32), jnp.cumsum(cpad)[:-1]])
    tile_start = (jnp.arange(NT) * TM).astype(jnp.float32)
    te = (jnp.sum(aoff[None, :] <= tile_start[:, None], axis=1) - 1
          ).astype(jnp.int32)

    dispatch, combine = _sc_kernels()
    pos_flat = pos.reshape(2 * N)
    xs = dispatch(pos_flat, x_flat)
    y = _ffn(te, xs, w1.astype(jnp.bfloat16), w2.astype(jnp.bfloat16))
    out = combine(y, pos_flat, tp[:, 0], tp[:, 1])

    return (out.reshape(Bb, Tt, Dd), aux.reshape(()),
            probs.reshape(Bb, Tt, E), ti.reshape(Bb, Tt, K),
            tp.reshape(Bb, Tt, K))


# speedup vs baseline: 1.2195x; 1.2195x over previous
import functools

import jax
import jax.numpy as jnp
from jax import lax
from jax.experimental import pallas as pl
from jax.experimental.pallas import tpu as pltpu
from jax.experimental.pallas import tpu_sc as plsc

N = 2048
D = 1024
E = 8
K = 2
F = 2048
TM = 128
NB = N * K + E * TM
NT = NB // TM
LANES = 128
NW = 32


def _router_body(x_ref, gw_ref, probs_ref, ti_ref, tp_ref, aux_ref,
                 pos_ref, counts_ref):
    x = x_ref[...]
    gw = gw_ref[...]
    logits = jnp.dot(x, gw, preferred_element_type=jnp.float32)

    lane = lax.broadcasted_iota(jnp.int32, (N, LANES), 1)
    valid = lane < E
    neg = jnp.float32(-1e30)
    lm = jnp.where(valid, logits, neg)
    m = jnp.max(lm, axis=1, keepdims=True)
    ex = jnp.where(valid, jnp.exp(lm - m), 0.0)
    s = jnp.sum(ex, axis=1, keepdims=True)
    p = ex / s
    probs_ref[...] = p[:, :E]

    big = jnp.int32(999)
    v0 = jnp.max(p, axis=1, keepdims=True)
    i0 = jnp.min(jnp.where((p == v0) & valid, lane, big), axis=1,
                 keepdims=True)
    p1 = jnp.where(valid & (lane != i0), p, -1.0)
    v1 = jnp.max(p1, axis=1, keepdims=True)
    i1 = jnp.min(jnp.where(p1 == v1, lane, big), axis=1, keepdims=True)
    ti_ref[...] = jnp.concatenate([i0, i1], axis=1)
    s2 = v0 + v1
    tp_ref[...] = jnp.concatenate([v0 / s2, v1 / s2], axis=1)

    ohA = jnp.where(lane == i0, 1.0, 0.0)
    ohB = jnp.where(lane == i1, 1.0, 0.0)
    cnt = jnp.sum(ohA + ohB, axis=0, keepdims=True)
    sp = jnp.sum(p, axis=0, keepdims=True)
    aux_ref[...] = (E / (N * N)) * jnp.sum(cnt * sp, axis=1, keepdims=True)

    r = lax.broadcasted_iota(jnp.int32, (TM, TM), 0)
    c = lax.broadcasted_iota(jnp.int32, (TM, TM), 1)
    tstrict = (r > c).astype(jnp.float32)
    mlt = (r < c).astype(jnp.float32)

    oh = jnp.concatenate([ohA, ohB], axis=0)
    carry = jnp.zeros((1, LANES), jnp.float32)
    rank_blocks = []
    for b in range(2 * N // TM):
        blk = lax.slice(oh, (b * TM, 0), ((b + 1) * TM, LANES))
        rank_blocks.append(
            jnp.dot(tstrict, blk, preferred_element_type=jnp.float32) + carry)
        carry = carry + jnp.sum(blk, axis=0, keepdims=True)
    ranks = jnp.concatenate(rank_blocks, axis=0)
    counts = carry
    cpad = jnp.ceil(counts / TM) * TM
    aoff = jnp.dot(cpad, mlt, preferred_element_type=jnp.float32)
    posf = jnp.sum(oh * (ranks + aoff), axis=1, keepdims=True)
    pos_ref[...] = posf.astype(jnp.int32)
    counts_ref[...] = counts


_router = pl.pallas_call(
    _router_body,
    out_shape=(
        jax.ShapeDtypeStruct((N, E), jnp.float32),
        jax.ShapeDtypeStruct((N, K), jnp.int32),
        jax.ShapeDtypeStruct((N, K), jnp.float32),
        jax.ShapeDtypeStruct((1, 1), jnp.float32),
        jax.ShapeDtypeStruct((2 * N, 1), jnp.int32),
        jax.ShapeDtypeStruct((1, LANES), jnp.float32),
    ),
)


_RPW = NB // NW
_ZPW = NB // 16
_PPS = 2 * N // 16


def _dispatch_body(pos2_hbm, x_hbm, xs_hbm, zbuf, pslice, vbuf, myids,
                   rb0, rb1, ts_sh, semg):
    s = lax.axis_index("s")
    c = lax.axis_index("c")

    def zb(i, carry):
        zbuf[pl.ds(i * 16, 16)] = jnp.zeros((16,), jnp.int32)
        return carry
    lax.fori_loop(0, _ZPW // 16, zb, 0)
    pltpu.sync_copy(zbuf, ts_sh.at[pl.ds(s * _ZPW, _ZPW)])

    pltpu.sync_copy(pos2_hbm.at[pl.ds(s * 2, 2)], pslice)
    for j in range(2):
        def vb(i, carry, j=j):
            vbuf[j, pl.ds(i * 16, 16)] = (
                (s * _PPS + j * 128 + i * 16 + lax.iota(jnp.int32, 16))
                & (N - 1))
            return carry
        lax.fori_loop(0, 8, vb, 0)
    plsc.subcore_barrier()

    for j in range(2):
        pltpu.sync_copy(vbuf.at[j], ts_sh.at[pslice.at[j]], add=True)
    plsc.subcore_barrier()

    wid = s * 2 + c
    base = wid * _RPW
    pltpu.sync_copy(ts_sh.at[pl.ds(base, _RPW)], myids)
    nch = _RPW // 32
    cp = pltpu.async_copy(x_hbm.at[myids.at[pl.ds(0, 32)]], rb0, semg)
    for ch in range(nch):
        rb = rb0 if ch % 2 == 0 else rb1
        nrb = rb1 if ch % 2 == 0 else rb0
        cp.wait()
        if ch + 1 < nch:
            cp = pltpu.async_copy(
                x_hbm.at[myids.at[pl.ds((ch + 1) * 32, 32)]], nrb, semg)
        pltpu.sync_copy(rb, xs_hbm.at[pl.ds(base + ch * 32, 32)])


def _ffn_body(te_ref, xs_ref, w1_ref, w2_ref, y_ref):
    @pl.when(pl.program_id(0) < te_ref[NT])
    def _():
        h = jnp.dot(xs_ref[...], w1_ref[0],
                    preferred_element_type=jnp.float32)
        h = 0.5 * h * (1.0 + lax.erf(h * 0.7071067811865476))
        y_ref[...] = jnp.dot(h, w2_ref[0],
                             preferred_element_type=jnp.float32)


_ffn = pl.pallas_call(
    _ffn_body,
    grid_spec=pltpu.PrefetchScalarGridSpec(
        num_scalar_prefetch=1,
        grid=(NT,),
        in_specs=[
            pl.BlockSpec((TM, D), lambda t, te: (t, 0)),
            pl.BlockSpec((1, D, F), lambda t, te: (te[t], 0, 0)),
            pl.BlockSpec((1, F, D), lambda t, te: (te[t], 0, 0)),
        ],
        out_specs=pl.BlockSpec((TM, D), lambda t, te: (t, 0)),
    ),
    out_shape=jax.ShapeDtypeStruct((NB, D), jnp.float32),
    compiler_params=pltpu.CompilerParams(
        dimension_semantics=("arbitrary",)),
)


_TPW = N // NW


_CCH = _TPW // 16


def _combine_body(y_hbm, pos_hbm, tpa_hbm, tpb_hbm, out_hbm, ia0, ib0,
                  ia1, ib1, ta_v, tb_v, ba0, bb0, ba1, bb1, obuf,
                  sema, semb):
    s = lax.axis_index("s")
    c = lax.axis_index("c")
    wid = s * 2 + c
    base = wid * _TPW

    def load_idx(ch, ia, ib):
        t0 = base + ch * 16
        pltpu.sync_copy(pos_hbm.at[pl.ds(t0, 16)], ia)
        pltpu.sync_copy(pos_hbm.at[pl.ds(N + t0, 16)], ib)

    load_idx(0, ia0, ib0)
    ca = pltpu.async_copy(y_hbm.at[ia0], ba0, sema)
    cb = pltpu.async_copy(y_hbm.at[ib0], bb0, semb)
    for ch in range(_CCH):
        par = ch % 2
        ba, bb = (ba0, bb0) if par == 0 else (ba1, bb1)
        nba, nbb = (ba1, bb1) if par == 0 else (ba0, bb0)
        nia, nib = (ia1, ib1) if par == 0 else (ia0, ib0)
        t0 = base + ch * 16
        pltpu.sync_copy(tpa_hbm.at[pl.ds(t0, 16)], ta_v)
        pltpu.sync_copy(tpb_hbm.at[pl.ds(t0, 16)], tb_v)
        ca.wait()
        cb.wait()
        if ch + 1 < _CCH:
            load_idx(ch + 1, nia, nib)
            ca = pltpu.async_copy(y_hbm.at[nia], nba, sema)
            cb = pltpu.async_copy(y_hbm.at[nib], nbb, semb)
        tav = ta_v[...]
        tbv = tb_v[...]
        for i in range(16):
            a = tav[i]
            b = tbv[i]

            def row_body(j, carry, i=i, a=a, b=b, ba=ba, bb=bb):
                for jj in range(4):
                    sl = pl.ds(j * 64 + jj * 16, 16)
                    obuf[i, sl] = ba[i, sl] * a + bb[i, sl] * b
                return carry
            lax.fori_loop(0, D // 64, row_body, 0)
        pltpu.sync_copy(obuf, out_hbm.at[pl.ds(t0, 16)])


@functools.cache
def _sc_kernels():
    mesh = plsc.VectorSubcoreMesh(core_axis_name="c", subcore_axis_name="s")
    dispatch = pl.kernel(
        _dispatch_body,
        mesh=mesh,
        out_type=jax.ShapeDtypeStruct((NB, D), jnp.float32),
        scratch_types=[
            pltpu.VMEM((_ZPW,), jnp.int32),
            pltpu.VMEM((2, 128), jnp.int32),
            pltpu.VMEM((2, 128), jnp.int32),
            pltpu.VMEM((_RPW,), jnp.int32),
            pltpu.VMEM((32, D), jnp.float32),
            pltpu.VMEM((32, D), jnp.float32),
            pltpu.VMEM_SHARED((NB,), jnp.int32),
            pltpu.SemaphoreType.DMA,
        ],
        compiler_params=pltpu.CompilerParams(needs_layout_passes=False),
    )
    combine = pl.kernel(
        _combine_body,
        mesh=mesh,
        out_type=jax.ShapeDtypeStruct((N, D), jnp.float32),
        scratch_types=[
            pltpu.VMEM((16,), jnp.int32),
            pltpu.VMEM((16,), jnp.int32),
            pltpu.VMEM((16,), jnp.int32),
            pltpu.VMEM((16,), jnp.int32),
            pltpu.VMEM((16,), jnp.float32),
            pltpu.VMEM((16,), jnp.float32),
            pltpu.VMEM((16, D), jnp.float32),
            pltpu.VMEM((16, D), jnp.float32),
            pltpu.VMEM((16, D), jnp.float32),
            pltpu.VMEM((16, D), jnp.float32),
            pltpu.VMEM((16, D), jnp.float32),
            pltpu.SemaphoreType.DMA,
            pltpu.SemaphoreType.DMA,
        ],
    )
    return dispatch, combine


def kernel(x, gate_w, w1, w2):
    Bb, Tt, Dd = x.shape
    x_flat = x.reshape(N, D)
    gwp = jnp.pad(gate_w, ((0, 0), (0, LANES - E)))
    probs, ti, tp, aux, pos, counts = _router(x_flat, gwp)

    counts8 = counts[0, :E]
    cpad = jnp.ceil(counts8 / TM) * TM
    aoff = jnp.concatenate(
        [jnp.zeros((1,), jnp.float32), jnp.cumsum(cpad)[:-1]])
    tile_start = (jnp.arange(NT) * TM).astype(jnp.float32)
    te = (jnp.sum(aoff[None, :] <= tile_start[:, None], axis=1) - 1
          ).astype(jnp.int32)
    used = (jnp.sum(cpad) / TM).astype(jnp.int32)
    te_ext = jnp.concatenate([te, used[None]])

    dispatch, combine = _sc_kernels()
    pos_flat = pos.reshape(2 * N)
    xs = dispatch(pos_flat.reshape(32, 128), x_flat)
    y = _ffn(te_ext, xs, w1, w2)
    out = combine(y, pos_flat, tp[:, 0], tp[:, 1])

    return (out.reshape(Bb, Tt, Dd), aux.reshape(()),
            probs.reshape(Bb, Tt, E), ti.reshape(Bb, Tt, K),
            tp.reshape(Bb, Tt, K))


# speedup vs baseline: 1.2239x; 1.0036x over previous
import functools

import jax
import jax.numpy as jnp
from jax import lax
from jax.experimental import pallas as pl
from jax.experimental.pallas import tpu as pltpu
from jax.experimental.pallas import tpu_sc as plsc

N = 2048
D = 1024
E = 8
K = 2
F = 2048
TM = 128
NB = N * K + E * TM
NT = NB // TM
LANES = 128
NW = 32


def _router_body(x_ref, gw_ref, probs_ref, ti_ref, tp_ref, aux_ref,
                 pos_ref, counts_ref):
    x = x_ref[...]
    gw = gw_ref[...]
    logits = jnp.dot(x, gw, preferred_element_type=jnp.float32)

    lane = lax.broadcasted_iota(jnp.int32, (N, LANES), 1)
    valid = lane < E
    neg = jnp.float32(-1e30)
    lm = jnp.where(valid, logits, neg)
    m = jnp.max(lm, axis=1, keepdims=True)
    ex = jnp.where(valid, jnp.exp(lm - m), 0.0)
    s = jnp.sum(ex, axis=1, keepdims=True)
    p = ex / s
    probs_ref[...] = p[:, :E]

    big = jnp.int32(999)
    v0 = jnp.max(p, axis=1, keepdims=True)
    i0 = jnp.min(jnp.where((p == v0) & valid, lane, big), axis=1,
                 keepdims=True)
    p1 = jnp.where(valid & (lane != i0), p, -1.0)
    v1 = jnp.max(p1, axis=1, keepdims=True)
    i1 = jnp.min(jnp.where(p1 == v1, lane, big), axis=1, keepdims=True)
    ti_ref[...] = jnp.concatenate([i0, i1], axis=1)
    s2 = v0 + v1
    tp_ref[...] = jnp.concatenate([v0 / s2, v1 / s2], axis=1)

    ohA = jnp.where(lane == i0, 1.0, 0.0)
    ohB = jnp.where(lane == i1, 1.0, 0.0)
    cnt = jnp.sum(ohA + ohB, axis=0, keepdims=True)
    sp = jnp.sum(p, axis=0, keepdims=True)
    aux_ref[...] = (E / (N * N)) * jnp.sum(cnt * sp, axis=1, keepdims=True)

    r = lax.broadcasted_iota(jnp.int32, (TM, TM), 0)
    c = lax.broadcasted_iota(jnp.int32, (TM, TM), 1)
    tstrict = (r > c).astype(jnp.float32)
    mlt = (r < c).astype(jnp.float32)

    oh = jnp.concatenate([ohA, ohB], axis=0)
    carry = jnp.zeros((1, LANES), jnp.float32)
    rank_blocks = []
    for b in range(2 * N // TM):
        blk = lax.slice(oh, (b * TM, 0), ((b + 1) * TM, LANES))
        rank_blocks.append(
            jnp.dot(tstrict, blk, preferred_element_type=jnp.float32) + carry)
        carry = carry + jnp.sum(blk, axis=0, keepdims=True)
    ranks = jnp.concatenate(rank_blocks, axis=0)
    counts = carry
    cpad = jnp.ceil(counts / TM) * TM
    aoff = jnp.dot(cpad, mlt, preferred_element_type=jnp.float32)
    posf = jnp.sum(oh * (ranks + aoff), axis=1, keepdims=True)
    pos_ref[...] = posf.astype(jnp.int32)
    counts_ref[...] = counts


_router = pl.pallas_call(
    _router_body,
    out_shape=(
        jax.ShapeDtypeStruct((N, E), jnp.float32),
        jax.ShapeDtypeStruct((N, K), jnp.int32),
        jax.ShapeDtypeStruct((N, K), jnp.float32),
        jax.ShapeDtypeStruct((1, 1), jnp.float32),
        jax.ShapeDtypeStruct((2 * N, 1), jnp.int32),
        jax.ShapeDtypeStruct((1, LANES), jnp.float32),
    ),
)


_RPW = NB // NW
_ZPW = NB // 16
_PPS = 2 * N // 16


def _dispatch_body(pos2_hbm, x_hbm, xs_hbm, zbuf, pslice, vbuf, myids,
                   rb0, rb1, rb2, ts_sh, semg, sems):
    s = lax.axis_index("s")
    c = lax.axis_index("c")

    def zb(i, carry):
        zbuf[pl.ds(i * 16, 16)] = jnp.zeros((16,), jnp.int32)
        return carry
    lax.fori_loop(0, _ZPW // 16, zb, 0)
    pltpu.sync_copy(zbuf, ts_sh.at[pl.ds(s * _ZPW, _ZPW)])

    pltpu.sync_copy(pos2_hbm.at[pl.ds(s * 2, 2)], pslice)
    for j in range(2):
        def vb(i, carry, j=j):
            vbuf[j, pl.ds(i * 16, 16)] = (
                (s * _PPS + j * 128 + i * 16 + lax.iota(jnp.int32, 16))
                & (N - 1))
            return carry
        lax.fori_loop(0, 8, vb, 0)
    plsc.subcore_barrier()

    for j in range(2):
        pltpu.sync_copy(vbuf.at[j], ts_sh.at[pslice.at[j]], add=True)
    plsc.subcore_barrier()

    wid = s * 2 + c
    base = wid * _RPW
    pltpu.sync_copy(ts_sh.at[pl.ds(base, _RPW)], myids)
    nch = _RPW // 32
    rbs = (rb0, rb1, rb2)
    g = [None, None, None]
    st = [None, None, None]
    for ch in range(min(3, nch)):
        g[ch] = pltpu.async_copy(
            x_hbm.at[myids.at[pl.ds(ch * 32, 32)]], rbs[ch], semg)
    for ch in range(nch):
        b = ch % 3
        g[b].wait()
        st[b] = pltpu.async_copy(
            rbs[b], xs_hbm.at[pl.ds(base + ch * 32, 32)], sems)
        if ch + 3 < nch:
            st[b].wait()
            g[b] = pltpu.async_copy(
                x_hbm.at[myids.at[pl.ds((ch + 3) * 32, 32)]], rbs[b], semg)
    for ch in range(max(0, nch - 3), nch):
        if st[ch % 3] is not None:
            st[ch % 3].wait()
            st[ch % 3] = None


def _ffn_body(te_ref, xs_ref, w1_ref, w2_ref, y_ref):
    @pl.when(pl.program_id(0) < te_ref[NT])
    def _():
        h = jnp.dot(xs_ref[...], w1_ref[0],
                    preferred_element_type=jnp.float32)
        h = 0.5 * h * (1.0 + lax.erf(h * 0.7071067811865476))
        y_ref[...] = jnp.dot(h, w2_ref[0],
                             preferred_element_type=jnp.float32)


_ffn = pl.pallas_call(
    _ffn_body,
    grid_spec=pltpu.PrefetchScalarGridSpec(
        num_scalar_prefetch=1,
        grid=(NT,),
        in_specs=[
            pl.BlockSpec((TM, D), lambda t, te: (t, 0)),
            pl.BlockSpec((1, D, F), lambda t, te: (te[t], 0, 0)),
            pl.BlockSpec((1, F, D), lambda t, te: (te[t], 0, 0)),
        ],
        out_specs=pl.BlockSpec((TM, D), lambda t, te: (t, 0)),
    ),
    out_shape=jax.ShapeDtypeStruct((NB, D), jnp.float32),
    compiler_params=pltpu.CompilerParams(
        dimension_semantics=("arbitrary",)),
)


_TPW = N // NW


_CCH = _TPW // 16


def _combine_body(y_hbm, pos_hbm, tpa_hbm, tpb_hbm, out_hbm, ia0, ib0,
                  ia1, ib1, ta_v, tb_v, ba0, bb0, ba1, bb1, ob0, ob1,
                  sema, semb, semo):
    s = lax.axis_index("s")
    c = lax.axis_index("c")
    wid = s * 2 + c
    base = wid * _TPW

    def load_idx(ch, ia, ib):
        t0 = base + ch * 16
        pltpu.sync_copy(pos_hbm.at[pl.ds(t0, 16)], ia)
        pltpu.sync_copy(pos_hbm.at[pl.ds(N + t0, 16)], ib)

    load_idx(0, ia0, ib0)
    ca = pltpu.async_copy(y_hbm.at[ia0], ba0, sema)
    cb = pltpu.async_copy(y_hbm.at[ib0], bb0, semb)
    so = [None, None]
    for ch in range(_CCH):
        par = ch % 2
        ba, bb = (ba0, bb0) if par == 0 else (ba1, bb1)
        nba, nbb = (ba1, bb1) if par == 0 else (ba0, bb0)
        nia, nib = (ia1, ib1) if par == 0 else (ia0, ib0)
        obuf = ob0 if par == 0 else ob1
        t0 = base + ch * 16
        pltpu.sync_copy(tpa_hbm.at[pl.ds(t0, 16)], ta_v)
        pltpu.sync_copy(tpb_hbm.at[pl.ds(t0, 16)], tb_v)
        ca.wait()
        cb.wait()
        if ch + 1 < _CCH:
            load_idx(ch + 1, nia, nib)
            ca = pltpu.async_copy(y_hbm.at[nia], nba, sema)
            cb = pltpu.async_copy(y_hbm.at[nib], nbb, semb)
        if so[par] is not None:
            so[par].wait()
        tav = ta_v[...]
        tbv = tb_v[...]
        for i in range(16):
            a = tav[i]
            b = tbv[i]

            def row_body(j, carry, i=i, a=a, b=b, ba=ba, bb=bb,
                         obuf=obuf):
                for jj in range(4):
                    sl = pl.ds(j * 64 + jj * 16, 16)
                    obuf[i, sl] = ba[i, sl] * a + bb[i, sl] * b
                return carry
            lax.fori_loop(0, D // 64, row_body, 0)
        so[par] = pltpu.async_copy(obuf, out_hbm.at[pl.ds(t0, 16)], semo)
    for p in range(2):
        if so[p] is not None:
            so[p].wait()


@functools.cache
def _sc_kernels():
    mesh = plsc.VectorSubcoreMesh(core_axis_name="c", subcore_axis_name="s")
    dispatch = pl.kernel(
        _dispatch_body,
        mesh=mesh,
        out_type=jax.ShapeDtypeStruct((NB, D), jnp.float32),
        scratch_types=[
            pltpu.VMEM((_ZPW,), jnp.int32),
            pltpu.VMEM((2, 128), jnp.int32),
            pltpu.VMEM((2, 128), jnp.int32),
            pltpu.VMEM((_RPW,), jnp.int32),
            pltpu.VMEM((32, D), jnp.float32),
            pltpu.VMEM((32, D), jnp.float32),
            pltpu.VMEM((32, D), jnp.float32),
            pltpu.VMEM_SHARED((NB,), jnp.int32),
            pltpu.SemaphoreType.DMA,
            pltpu.SemaphoreType.DMA,
        ],
        compiler_params=pltpu.CompilerParams(needs_layout_passes=False),
    )
    combine = pl.kernel(
        _combine_body,
        mesh=mesh,
        out_type=jax.ShapeDtypeStruct((N, D), jnp.float32),
        scratch_types=[
            pltpu.VMEM((16,), jnp.int32),
            pltpu.VMEM((16,), jnp.int32),
            pltpu.VMEM((16,), jnp.int32),
            pltpu.VMEM((16,), jnp.int32),
            pltpu.VMEM((16,), jnp.float32),
            pltpu.VMEM((16,), jnp.float32),
            pltpu.VMEM((16, D), jnp.float32),
            pltpu.VMEM((16, D), jnp.float32),
            pltpu.VMEM((16, D), jnp.float32),
            pltpu.VMEM((16, D), jnp.float32),
            pltpu.VMEM((16, D), jnp.float32),
            pltpu.VMEM((16, D), jnp.float32),
            pltpu.SemaphoreType.DMA,
            pltpu.SemaphoreType.DMA,
            pltpu.SemaphoreType.DMA,
        ],
    )
    return dispatch, combine


def kernel(x, gate_w, w1, w2):
    Bb, Tt, Dd = x.shape
    x_flat = x.reshape(N, D)
    gwp = jnp.pad(gate_w, ((0, 0), (0, LANES - E)))
    probs, ti, tp, aux, pos, counts = _router(x_flat, gwp)

    counts8 = counts[0, :E]
    cpad = jnp.ceil(counts8 / TM) * TM
    aoff = jnp.concatenate(
        [jnp.zeros((1,), jnp.float32), jnp.cumsum(cpad)[:-1]])
    tile_start = (jnp.arange(NT) * TM).astype(jnp.float32)
    te = (jnp.sum(aoff[None, :] <= tile_start[:, None], axis=1) - 1
          ).astype(jnp.int32)
    used = (jnp.sum(cpad) / TM).astype(jnp.int32)
    te_ext = jnp.concatenate([te, used[None]])

    dispatch, combine = _sc_kernels()
    pos_flat = pos.reshape(2 * N)
    xs = dispatch(pos_flat.reshape(32, 128), x_flat)
    y = _ffn(te_ext, xs, w1, w2)
    out = combine(y, pos_flat, tp[:, 0], tp[:, 1])

    return (out.reshape(Bb, Tt, Dd), aux.reshape(()),
            probs.reshape(Bb, Tt, E), ti.reshape(Bb, Tt, K),
            tp.reshape(Bb, Tt, K))


# speedup vs baseline: 1.3294x; 1.0862x over previous
import functools

import jax
import jax.numpy as jnp
from jax import lax
from jax.experimental import pallas as pl
from jax.experimental.pallas import tpu as pltpu
from jax.experimental.pallas import tpu_sc as plsc

N = 2048
D = 1024
E = 8
K = 2
F = 2048
TM = 128
NB = N * K + E * TM
NT = NB // TM
LANES = 128
NW = 32


def _router_body(x_ref, gw_ref, probs_ref, ti_ref, tp_ref, aux_ref,
                 pos_ref, counts_ref):
    x = x_ref[...]
    gw = gw_ref[...]
    logits = jnp.dot(x, gw, preferred_element_type=jnp.float32)

    lane = lax.broadcasted_iota(jnp.int32, (N, LANES), 1)
    valid = lane < E
    neg = jnp.float32(-1e30)
    lm = jnp.where(valid, logits, neg)
    m = jnp.max(lm, axis=1, keepdims=True)
    ex = jnp.where(valid, jnp.exp(lm - m), 0.0)
    s = jnp.sum(ex, axis=1, keepdims=True)
    p = ex / s
    probs_ref[...] = p[:, :E]

    big = jnp.int32(999)
    v0 = jnp.max(p, axis=1, keepdims=True)
    i0 = jnp.min(jnp.where((p == v0) & valid, lane, big), axis=1,
                 keepdims=True)
    p1 = jnp.where(valid & (lane != i0), p, -1.0)
    v1 = jnp.max(p1, axis=1, keepdims=True)
    i1 = jnp.min(jnp.where(p1 == v1, lane, big), axis=1, keepdims=True)
    ti_ref[...] = jnp.concatenate([i0, i1], axis=1)
    s2 = v0 + v1
    tp_ref[...] = jnp.concatenate([v0 / s2, v1 / s2], axis=1)

    ohA = jnp.where(lane == i0, 1.0, 0.0)
    ohB = jnp.where(lane == i1, 1.0, 0.0)
    cnt = jnp.sum(ohA + ohB, axis=0, keepdims=True)
    sp = jnp.sum(p, axis=0, keepdims=True)
    aux_ref[...] = (E / (N * N)) * jnp.sum(cnt * sp, axis=1, keepdims=True)

    r = lax.broadcasted_iota(jnp.int32, (TM, TM), 0)
    c = lax.broadcasted_iota(jnp.int32, (TM, TM), 1)
    tstrict = (r > c).astype(jnp.float32)
    mlt = (r < c).astype(jnp.float32)

    oh = jnp.concatenate([ohA, ohB], axis=0)
    carry = jnp.zeros((1, LANES), jnp.float32)
    rank_blocks = []
    for b in range(2 * N // TM):
        blk = lax.slice(oh, (b * TM, 0), ((b + 1) * TM, LANES))
        rank_blocks.append(
            jnp.dot(tstrict, blk, preferred_element_type=jnp.float32) + carry)
        carry = carry + jnp.sum(blk, axis=0, keepdims=True)
    ranks = jnp.concatenate(rank_blocks, axis=0)
    counts = carry
    cpad = jnp.ceil(counts / TM) * TM
    aoff = jnp.dot(cpad, mlt, preferred_element_type=jnp.float32)
    posf = jnp.sum(oh * (ranks + aoff), axis=1, keepdims=True)
    pos_ref[...] = posf.astype(jnp.int32)
    counts_ref[...] = counts


_router = pl.pallas_call(
    _router_body,
    out_shape=(
        jax.ShapeDtypeStruct((N, E), jnp.float32),
        jax.ShapeDtypeStruct((N, K), jnp.int32),
        jax.ShapeDtypeStruct((N, K), jnp.float32),
        jax.ShapeDtypeStruct((1, 1), jnp.float32),
        jax.ShapeDtypeStruct((2 * N, 1), jnp.int32),
        jax.ShapeDtypeStruct((1, LANES), jnp.float32),
    ),
)


_RPW = NB // NW
_ZPW = NB // 16
_PPS = 2 * N // 16


def _dispatch_body(pos2_hbm, x_hbm, ur_hbm, xs_hbm, zbuf, pslice, vbuf,
                   myids, uv, rb0, rb1, rb2, ts_sh, semg, sems):
    s = lax.axis_index("s")
    c = lax.axis_index("c")

    def zb(i, carry):
        zbuf[pl.ds(i * 16, 16)] = jnp.zeros((16,), jnp.int32)
        return carry
    lax.fori_loop(0, _ZPW // 16, zb, 0)
    pltpu.sync_copy(zbuf, ts_sh.at[pl.ds(s * _ZPW, _ZPW)])

    pltpu.sync_copy(pos2_hbm.at[pl.ds(s * 2, 2)], pslice)
    for j in range(2):
        def vb(i, carry, j=j):
            vbuf[j, pl.ds(i * 16, 16)] = (
                (s * _PPS + j * 128 + i * 16 + lax.iota(jnp.int32, 16))
                & (N - 1))
            return carry
        lax.fori_loop(0, 8, vb, 0)
    plsc.subcore_barrier()

    for j in range(2):
        pltpu.sync_copy(vbuf.at[j], ts_sh.at[pslice.at[j]], add=True)
    plsc.subcore_barrier()

    wid = s * 2 + c
    base = wid * _RPW
    pltpu.sync_copy(ur_hbm, uv)
    nrows = uv[...][0]

    @pl.when(base < nrows)
    def _phase2():
        pltpu.sync_copy(ts_sh.at[pl.ds(base, _RPW)], myids)
        cw = 40
        nch = _RPW // cw
        rbs = (rb0, rb1, rb2)
        g = [None, None, None]
        st = [None, None, None]
        for ch in range(min(3, nch)):
            g[ch] = pltpu.async_copy(
                x_hbm.at[myids.at[pl.ds(ch * cw, cw)]], rbs[ch], semg)
        for ch in range(nch):
            b = ch % 3
            g[b].wait()
            st[b] = pltpu.async_copy(
                rbs[b], xs_hbm.at[pl.ds(base + ch * cw, cw)], sems)
            if ch + 3 < nch:
                st[b].wait()
                g[b] = pltpu.async_copy(
                    x_hbm.at[myids.at[pl.ds((ch + 3) * cw, cw)]],
                    rbs[b], semg)
        for ch in range(max(0, nch - 3), nch):
            if st[ch % 3] is not None:
                st[ch % 3].wait()
                st[ch % 3] = None


def _ffn_body(te_ref, xs_ref, w1_ref, w2_ref, y_ref):
    @pl.when(pl.program_id(0) < te_ref[NT])
    def _():
        h = jnp.dot(xs_ref[...], w1_ref[0],
                    preferred_element_type=jnp.float32)
        h = 0.5 * h * (1.0 + lax.erf(h * 0.7071067811865476))
        y_ref[...] = jnp.dot(h, w2_ref[0],
                             preferred_element_type=jnp.float32)


_ffn = pl.pallas_call(
    _ffn_body,
    grid_spec=pltpu.PrefetchScalarGridSpec(
        num_scalar_prefetch=1,
        grid=(NT,),
        in_specs=[
            pl.BlockSpec((TM, D), lambda t, te: (t, 0)),
            pl.BlockSpec((1, D, F), lambda t, te: (te[t], 0, 0)),
            pl.BlockSpec((1, F, D), lambda t, te: (te[t], 0, 0)),
        ],
        out_specs=pl.BlockSpec((TM, D), lambda t, te: (t, 0)),
    ),
    out_shape=jax.ShapeDtypeStruct((NB, D), jnp.float32),
    compiler_params=pltpu.CompilerParams(
        dimension_semantics=("arbitrary",)),
)


_TPW = N // NW


_CCH = _TPW // 16


def _combine_body(y_hbm, pos_hbm, tpa_hbm, tpb_hbm, out_hbm, ia0, ib0,
                  ia1, ib1, ta_v, tb_v, ba0, bb0, ba1, bb1, ob0, ob1,
                  sema, semb, semo):
    s = lax.axis_index("s")
    c = lax.axis_index("c")
    wid = s * 2 + c
    base = wid * _TPW

    def load_idx(ch, ia, ib):
        t0 = base + ch * 16
        pltpu.sync_copy(pos_hbm.at[pl.ds(t0, 16)], ia)
        pltpu.sync_copy(pos_hbm.at[pl.ds(N + t0, 16)], ib)

    load_idx(0, ia0, ib0)
    ca = pltpu.async_copy(y_hbm.at[ia0], ba0, sema)
    cb = pltpu.async_copy(y_hbm.at[ib0], bb0, semb)
    so = [None, None]
    for ch in range(_CCH):
        par = ch % 2
        ba, bb = (ba0, bb0) if par == 0 else (ba1, bb1)
        nba, nbb = (ba1, bb1) if par == 0 else (ba0, bb0)
        nia, nib = (ia1, ib1) if par == 0 else (ia0, ib0)
        obuf = ob0 if par == 0 else ob1
        t0 = base + ch * 16
        pltpu.sync_copy(tpa_hbm.at[pl.ds(t0, 16)], ta_v)
        pltpu.sync_copy(tpb_hbm.at[pl.ds(t0, 16)], tb_v)
        ca.wait()
        cb.wait()
        if ch + 1 < _CCH:
            load_idx(ch + 1, nia, nib)
            ca = pltpu.async_copy(y_hbm.at[nia], nba, sema)
            cb = pltpu.async_copy(y_hbm.at[nib], nbb, semb)
        if so[par] is not None:
            so[par].wait()
        tav = ta_v[...]
        tbv = tb_v[...]
        for i in range(16):
            a = tav[i]
            b = tbv[i]

            def row_body(j, carry, i=i, a=a, b=b, ba=ba, bb=bb,
                         obuf=obuf):
                for jj in range(4):
                    sl = pl.ds(j * 64 + jj * 16, 16)
                    obuf[i, sl] = ba[i, sl] * a + bb[i, sl] * b
                return carry
            lax.fori_loop(0, D // 64, row_body, 0)
        so[par] = pltpu.async_copy(obuf, out_hbm.at[pl.ds(t0, 16)], semo)
    for p in range(2):
        if so[p] is not None:
            so[p].wait()


@functools.cache
def _sc_kernels():
    mesh = plsc.VectorSubcoreMesh(core_axis_name="c", subcore_axis_name="s")
    dispatch = pl.kernel(
        _dispatch_body,
        mesh=mesh,
        out_type=jax.ShapeDtypeStruct((NB, D), jnp.float32),
        scratch_types=[
            pltpu.VMEM((_ZPW,), jnp.int32),
            pltpu.VMEM((2, 128), jnp.int32),
            pltpu.VMEM((2, 128), jnp.int32),
            pltpu.VMEM((_RPW,), jnp.int32),
            pltpu.VMEM((16,), jnp.int32),
            pltpu.VMEM((40, D), jnp.float32),
            pltpu.VMEM((40, D), jnp.float32),
            pltpu.VMEM((40, D), jnp.float32),
            pltpu.VMEM_SHARED((NB,), jnp.int32),
            pltpu.SemaphoreType.DMA,
            pltpu.SemaphoreType.DMA,
        ],
        compiler_params=pltpu.CompilerParams(needs_layout_passes=False),
    )
    combine = pl.kernel(
        _combine_body,
        mesh=mesh,
        out_type=jax.ShapeDtypeStruct((N, D), jnp.float32),
        scratch_types=[
            pltpu.VMEM((16,), jnp.int32),
            pltpu.VMEM((16,), jnp.int32),
            pltpu.VMEM((16,), jnp.int32),
            pltpu.VMEM((16,), jnp.int32),
            pltpu.VMEM((16,), jnp.float32),
            pltpu.VMEM((16,), jnp.float32),
            pltpu.VMEM((16, D), jnp.float32),
            pltpu.VMEM((16, D), jnp.float32),
            pltpu.VMEM((16, D), jnp.float32),
            pltpu.VMEM((16, D), jnp.float32),
            pltpu.VMEM((16, D), jnp.float32),
            pltpu.VMEM((16, D), jnp.float32),
            pltpu.SemaphoreType.DMA,
            pltpu.SemaphoreType.DMA,
            pltpu.SemaphoreType.DMA,
        ],
    )
    return dispatch, combine


def kernel(x, gate_w, w1, w2):
    Bb, Tt, Dd = x.shape
    x_flat = x.reshape(N, D)
    gwp = jnp.pad(gate_w, ((0, 0), (0, LANES - E)))
    probs, ti, tp, aux, pos, counts = _router(x_flat, gwp)

    counts8 = counts[0, :E]
    cpad = jnp.ceil(counts8 / TM) * TM
    aoff = jnp.concatenate(
        [jnp.zeros((1,), jnp.float32), jnp.cumsum(cpad)[:-1]])
    tile_start = (jnp.arange(NT) * TM).astype(jnp.float32)
    te = (jnp.sum(aoff[None, :] <= tile_start[:, None], axis=1) - 1
          ).astype(jnp.int32)
    used = (jnp.sum(cpad) / TM).astype(jnp.int32)
    te_ext = jnp.concatenate([te, used[None]])

    dispatch, combine = _sc_kernels()
    pos_flat = pos.reshape(2 * N)
    used_rows = jnp.full((16,), used * TM, jnp.int32)
    xs = dispatch(pos_flat.reshape(32, 128), x_flat, used_rows)
    y = _ffn(te_ext, xs, w1, w2)
    out = combine(y, pos_flat, tp[:, 0], tp[:, 1])

    return (out.reshape(Bb, Tt, Dd), aux.reshape(()),
            probs.reshape(Bb, Tt, E), ti.reshape(Bb, Tt, K),
            tp.reshape(Bb, Tt, K))


# speedup vs baseline: 1.3550x; 1.0193x over previous
import functools

import jax
import jax.numpy as jnp
from jax import lax
from jax.experimental import pallas as pl
from jax.experimental.pallas import tpu as pltpu
from jax.experimental.pallas import tpu_sc as plsc

N = 2048
D = 1024
E = 8
K = 2
F = 2048
TM = 128
NB = N * K + E * TM
NT = NB // TM
LANES = 128
NW = 32


def _router_body(x_ref, gw_ref, probs_ref, ti_ref, tp_ref, aux_ref,
                 pos_ref, counts_ref):
    x = x_ref[...]
    gw = gw_ref[...]
    logits = jnp.dot(x, gw, preferred_element_type=jnp.float32)

    lane = lax.broadcasted_iota(jnp.int32, (N, LANES), 1)
    valid = lane < E
    neg = jnp.float32(-1e30)
    lm = jnp.where(valid, logits, neg)
    m = jnp.max(lm, axis=1, keepdims=True)
    ex = jnp.where(valid, jnp.exp(lm - m), 0.0)
    s = jnp.sum(ex, axis=1, keepdims=True)
    p = ex / s
    probs_ref[...] = p[:, :E]

    big = jnp.int32(999)
    v0 = jnp.max(p, axis=1, keepdims=True)
    i0 = jnp.min(jnp.where((p == v0) & valid, lane, big), axis=1,
                 keepdims=True)
    p1 = jnp.where(valid & (lane != i0), p, -1.0)
    v1 = jnp.max(p1, axis=1, keepdims=True)
    i1 = jnp.min(jnp.where(p1 == v1, lane, big), axis=1, keepdims=True)
    ti_ref[...] = jnp.concatenate([i0, i1], axis=1)
    s2 = v0 + v1
    tp_ref[...] = jnp.concatenate([v0 / s2, v1 / s2], axis=1)

    ohA = jnp.where(lane == i0, 1.0, 0.0)
    ohB = jnp.where(lane == i1, 1.0, 0.0)
    cnt = jnp.sum(ohA + ohB, axis=0, keepdims=True)
    sp = jnp.sum(p, axis=0, keepdims=True)
    aux_ref[...] = (E / (N * N)) * jnp.sum(cnt * sp, axis=1, keepdims=True)

    r = lax.broadcasted_iota(jnp.int32, (TM, TM), 0)
    c = lax.broadcasted_iota(jnp.int32, (TM, TM), 1)
    tstrict = (r > c).astype(jnp.float32)
    mlt = (r < c).astype(jnp.float32)

    oh = jnp.concatenate([ohA, ohB], axis=0)
    carry = jnp.zeros((1, LANES), jnp.float32)
    rank_blocks = []
    for b in range(2 * N // TM):
        blk = lax.slice(oh, (b * TM, 0), ((b + 1) * TM, LANES))
        rank_blocks.append(
            jnp.dot(tstrict, blk, preferred_element_type=jnp.float32) + carry)
        carry = carry + jnp.sum(blk, axis=0, keepdims=True)
    ranks = jnp.concatenate(rank_blocks, axis=0)
    counts = carry
    cpad = jnp.ceil(counts / TM) * TM
    aoff = jnp.dot(cpad, mlt, preferred_element_type=jnp.float32)
    posf = jnp.sum(oh * (ranks + aoff), axis=1, keepdims=True)
    pos_ref[...] = posf.astype(jnp.int32)
    counts_ref[...] = counts


_router = pl.pallas_call(
    _router_body,
    out_shape=(
        jax.ShapeDtypeStruct((N, E), jnp.float32),
        jax.ShapeDtypeStruct((N, K), jnp.int32),
        jax.ShapeDtypeStruct((N, K), jnp.float32),
        jax.ShapeDtypeStruct((1, 1), jnp.float32),
        jax.ShapeDtypeStruct((2 * N, 1), jnp.int32),
        jax.ShapeDtypeStruct((1, LANES), jnp.float32),
    ),
)


_RPW = NB // NW
_ZPW = NB // 16
_PPS = 2 * N // 16


def _dispatch_body(pos2_hbm, x_hbm, ur_hbm, xs_hbm, zbuf, pslice, vbuf,
                   myids, uv, rb0, rb1, rb2, ts_sh, semg, sems):
    s = lax.axis_index("s")
    c = lax.axis_index("c")

    def zb(i, carry):
        zbuf[pl.ds(i * 16, 16)] = jnp.zeros((16,), jnp.int32)
        return carry
    lax.fori_loop(0, _ZPW // 16, zb, 0)
    pltpu.sync_copy(zbuf, ts_sh.at[pl.ds(s * _ZPW, _ZPW)])

    pltpu.sync_copy(pos2_hbm.at[pl.ds(s * 2, 2)], pslice)
    for j in range(2):
        def vb(i, carry, j=j):
            vbuf[j, pl.ds(i * 16, 16)] = (
                (s * _PPS + j * 128 + i * 16 + lax.iota(jnp.int32, 16))
                & (N - 1))
            return carry
        lax.fori_loop(0, 8, vb, 0)
    plsc.subcore_barrier()

    for j in range(2):
        pltpu.sync_copy(vbuf.at[j], ts_sh.at[pslice.at[j]], add=True)
    plsc.subcore_barrier()

    wid = s * 2 + c
    base = wid * _RPW
    pltpu.sync_copy(ur_hbm, uv)
    nrows = uv[...][0]

    @pl.when(base < nrows)
    def _phase2():
        pltpu.sync_copy(ts_sh.at[pl.ds(base, _RPW)], myids)
        cw = 40
        nch = _RPW // cw
        rbs = (rb0, rb1, rb2)
        g = [None, None, None]
        st = [None, None, None]
        for ch in range(min(3, nch)):
            g[ch] = pltpu.async_copy(
                x_hbm.at[myids.at[pl.ds(ch * cw, cw)]], rbs[ch], semg)
        for ch in range(nch):
            b = ch % 3
            g[b].wait()
            st[b] = pltpu.async_copy(
                rbs[b], xs_hbm.at[pl.ds(base + ch * cw, cw)], sems)
            if ch + 3 < nch:
                st[b].wait()
                g[b] = pltpu.async_copy(
                    x_hbm.at[myids.at[pl.ds((ch + 3) * cw, cw)]],
                    rbs[b], semg)
        for ch in range(max(0, nch - 3), nch):
            if st[ch % 3] is not None:
                st[ch % 3].wait()
                st[ch % 3] = None


def _ffn_body(te_ref, xs_ref, w1_ref, w2_ref, y_ref):
    @pl.when(pl.program_id(0) < te_ref[NT])
    def _():
        h = jnp.dot(xs_ref[...], w1_ref[0],
                    preferred_element_type=jnp.float32)
        h = 0.5 * h * (1.0 + lax.erf(h * 0.7071067811865476))
        y_ref[...] = jnp.dot(h, w2_ref[0],
                             preferred_element_type=jnp.float32)


_ffn = pl.pallas_call(
    _ffn_body,
    grid_spec=pltpu.PrefetchScalarGridSpec(
        num_scalar_prefetch=1,
        grid=(NT,),
        in_specs=[
            pl.BlockSpec((TM, D), lambda t, te: (t, 0)),
            pl.BlockSpec((1, D, F), lambda t, te: (te[t], 0, 0)),
            pl.BlockSpec((1, F, D), lambda t, te: (te[t], 0, 0)),
        ],
        out_specs=pl.BlockSpec((TM, D), lambda t, te: (t, 0)),
    ),
    out_shape=jax.ShapeDtypeStruct((NB, D), jnp.float32),
    compiler_params=pltpu.CompilerParams(
        dimension_semantics=("arbitrary",)),
)


_TPW = N // NW


_CCH = _TPW // 16


def _combine_body(y_hbm, pos_hbm, tpa_hbm, tpb_hbm, out_hbm, iaall,
                  iball, taall, tball, ba0, bb0, ba1, bb1, ob0, ob1,
                  sema, semb, semo):
    s = lax.axis_index("s")
    c = lax.axis_index("c")
    wid = s * 2 + c
    base = wid * _TPW

    pltpu.sync_copy(pos_hbm.at[pl.ds(base, _TPW)], iaall)
    pltpu.sync_copy(pos_hbm.at[pl.ds(N + base, _TPW)], iball)
    pltpu.sync_copy(tpa_hbm.at[pl.ds(base, _TPW)], taall)
    pltpu.sync_copy(tpb_hbm.at[pl.ds(base, _TPW)], tball)

    ca = pltpu.async_copy(y_hbm.at[iaall.at[pl.ds(0, 16)]], ba0, sema)
    cb = pltpu.async_copy(y_hbm.at[iball.at[pl.ds(0, 16)]], bb0, semb)
    so = [None, None]
    for ch in range(_CCH):
        par = ch % 2
        ba, bb = (ba0, bb0) if par == 0 else (ba1, bb1)
        nba, nbb = (ba1, bb1) if par == 0 else (ba0, bb0)
        obuf = ob0 if par == 0 else ob1
        t0 = base + ch * 16
        ca.wait()
        cb.wait()
        if ch + 1 < _CCH:
            nsl = pl.ds((ch + 1) * 16, 16)
            ca = pltpu.async_copy(y_hbm.at[iaall.at[nsl]], nba, sema)
            cb = pltpu.async_copy(y_hbm.at[iball.at[nsl]], nbb, semb)
        if so[par] is not None:
            so[par].wait()
        tav = taall[pl.ds(ch * 16, 16)]
        tbv = tball[pl.ds(ch * 16, 16)]
        for i in range(16):
            a = tav[i]
            b = tbv[i]

            def row_body(j, carry, i=i, a=a, b=b, ba=ba, bb=bb,
                         obuf=obuf):
                for jj in range(4):
                    sl = pl.ds(j * 64 + jj * 16, 16)
                    obuf[i, sl] = ba[i, sl] * a + bb[i, sl] * b
                return carry
            lax.fori_loop(0, D // 64, row_body, 0)
        so[par] = pltpu.async_copy(obuf, out_hbm.at[pl.ds(t0, 16)], semo)
    for p in range(2):
        if so[p] is not None:
            so[p].wait()


@functools.cache
def _sc_kernels():
    mesh = plsc.VectorSubcoreMesh(core_axis_name="c", subcore_axis_name="s")
    dispatch = pl.kernel(
        _dispatch_body,
        mesh=mesh,
        out_type=jax.ShapeDtypeStruct((NB, D), jnp.float32),
        scratch_types=[
            pltpu.VMEM((_ZPW,), jnp.int32),
            pltpu.VMEM((2, 128), jnp.int32),
            pltpu.VMEM((2, 128), jnp.int32),
            pltpu.VMEM((_RPW,), jnp.int32),
            pltpu.VMEM((16,), jnp.int32),
            pltpu.VMEM((40, D), jnp.float32),
            pltpu.VMEM((40, D), jnp.float32),
            pltpu.VMEM((40, D), jnp.float32),
            pltpu.VMEM_SHARED((NB,), jnp.int32),
            pltpu.SemaphoreType.DMA,
            pltpu.SemaphoreType.DMA,
        ],
        compiler_params=pltpu.CompilerParams(needs_layout_passes=False),
    )
    combine = pl.kernel(
        _combine_body,
        mesh=mesh,
        out_type=jax.ShapeDtypeStruct((N, D), jnp.float32),
        scratch_types=[
            pltpu.VMEM((_TPW,), jnp.int32),
            pltpu.VMEM((_TPW,), jnp.int32),
            pltpu.VMEM((_TPW,), jnp.float32),
            pltpu.VMEM((_TPW,), jnp.float32),
            pltpu.VMEM((16, D), jnp.float32),
            pltpu.VMEM((16, D), jnp.float32),
            pltpu.VMEM((16, D), jnp.float32),
            pltpu.VMEM((16, D), jnp.float32),
            pltpu.VMEM((16, D), jnp.float32),
            pltpu.VMEM((16, D), jnp.float32),
            pltpu.SemaphoreType.DMA,
            pltpu.SemaphoreType.DMA,
            pltpu.SemaphoreType.DMA,
        ],
    )
    return dispatch, combine


def kernel(x, gate_w, w1, w2):
    Bb, Tt, Dd = x.shape
    x_flat = x.reshape(N, D)
    gwp = jnp.pad(gate_w, ((0, 0), (0, LANES - E)))
    probs, ti, tp, aux, pos, counts = _router(x_flat, gwp)

    counts8 = counts[0, :E]
    cpad = jnp.ceil(counts8 / TM) * TM
    aoff = jnp.concatenate(
        [jnp.zeros((1,), jnp.float32), jnp.cumsum(cpad)[:-1]])
    tile_start = (jnp.arange(NT) * TM).astype(jnp.float32)
    te = (jnp.sum(aoff[None, :] <= tile_start[:, None], axis=1) - 1
          ).astype(jnp.int32)
    used = (jnp.sum(cpad) / TM).astype(jnp.int32)
    te_ext = jnp.concatenate([te, used[None]])

    dispatch, combine = _sc_kernels()
    pos_flat = pos.reshape(2 * N)
    used_rows = jnp.full((16,), used * TM, jnp.int32)
    xs = dispatch(pos_flat.reshape(32, 128), x_flat, used_rows)
    y = _ffn(te_ext, xs, w1, w2)
    out = combine(y, pos_flat, tp[:, 0], tp[:, 1])

    return (out.reshape(Bb, Tt, Dd), aux.reshape(()),
            probs.reshape(Bb, Tt, E), ti.reshape(Bb, Tt, K),
            tp.reshape(Bb, Tt, K))


# speedup vs baseline: 1.3620x; 1.0051x over previous
import functools

import jax
import jax.numpy as jnp
from jax import lax
from jax.experimental import pallas as pl
from jax.experimental.pallas import tpu as pltpu
from jax.experimental.pallas import tpu_sc as plsc

N = 2048
D = 1024
E = 8
K = 2
F = 2048
TM = 128
NB = N * K + E * TM
NT = NB // TM
LANES = 128
NW = 32


def _router_body(x_ref, gw_ref, probs_ref, ti_ref, tp_ref, aux_ref,
                 pos_ref, te_ref, ur_ref):
    x = x_ref[...]
    gw = gw_ref[...]
    logits = jnp.dot(x, gw, preferred_element_type=jnp.float32)

    lane = lax.broadcasted_iota(jnp.int32, (N, LANES), 1)
    valid = lane < E
    neg = jnp.float32(-1e30)
    lm = jnp.where(valid, logits, neg)
    m = jnp.max(lm, axis=1, keepdims=True)
    ex = jnp.where(valid, jnp.exp(lm - m), 0.0)
    s = jnp.sum(ex, axis=1, keepdims=True)
    p = ex / s
    probs_ref[...] = p[:, :E]

    big = jnp.int32(999)
    v0 = jnp.max(p, axis=1, keepdims=True)
    i0 = jnp.min(jnp.where((p == v0) & valid, lane, big), axis=1,
                 keepdims=True)
    p1 = jnp.where(valid & (lane != i0), p, -1.0)
    v1 = jnp.max(p1, axis=1, keepdims=True)
    i1 = jnp.min(jnp.where(p1 == v1, lane, big), axis=1, keepdims=True)
    ti_ref[...] = jnp.concatenate([i0, i1], axis=1)
    s2 = v0 + v1
    tp_ref[...] = jnp.concatenate([v0 / s2, v1 / s2], axis=1)

    ohA = jnp.where(lane == i0, 1.0, 0.0)
    ohB = jnp.where(lane == i1, 1.0, 0.0)
    cnt = jnp.sum(ohA + ohB, axis=0, keepdims=True)
    sp = jnp.sum(p, axis=0, keepdims=True)
    aux_ref[...] = (E / (N * N)) * jnp.sum(cnt * sp, axis=1, keepdims=True)

    r = lax.broadcasted_iota(jnp.int32, (TM, TM), 0)
    c = lax.broadcasted_iota(jnp.int32, (TM, TM), 1)
    tstrict = (r > c).astype(jnp.float32)
    mlt = (r < c).astype(jnp.float32)

    oh = jnp.concatenate([ohA, ohB], axis=0)
    carry = jnp.zeros((1, LANES), jnp.float32)
    rank_blocks = []
    for b in range(2 * N // TM):
        blk = lax.slice(oh, (b * TM, 0), ((b + 1) * TM, LANES))
        rank_blocks.append(
            jnp.dot(tstrict, blk, preferred_element_type=jnp.float32) + carry)
        carry = carry + jnp.sum(blk, axis=0, keepdims=True)
    ranks = jnp.concatenate(rank_blocks, axis=0)
    counts = carry
    cpad = jnp.ceil(counts / TM) * TM
    aoff = jnp.dot(cpad, mlt, preferred_element_type=jnp.float32)
    posf = jnp.sum(oh * (ranks + aoff), axis=1, keepdims=True)
    pos_ref[...] = posf.astype(jnp.int32)

    valid8 = (c < E)
    pred = ((aoff <= (r * TM).astype(jnp.float32)) & valid8
            ).astype(jnp.float32)
    te_f = jnp.sum(pred, axis=1, keepdims=True) - 1.0
    used = jnp.sum(cpad, axis=1, keepdims=True) / TM
    te_full = jnp.where(r[:, :1] == NT, used, te_f)
    te_ref[...] = te_full.astype(jnp.int32)
    ur_ref[...] = jnp.broadcast_to(used * TM, (1, LANES)).astype(jnp.int32)


_router = pl.pallas_call(
    _router_body,
    out_shape=(
        jax.ShapeDtypeStruct((N, E), jnp.float32),
        jax.ShapeDtypeStruct((N, K), jnp.int32),
        jax.ShapeDtypeStruct((N, K), jnp.float32),
        jax.ShapeDtypeStruct((1, 1), jnp.float32),
        jax.ShapeDtypeStruct((2 * N, 1), jnp.int32),
        jax.ShapeDtypeStruct((TM, 1), jnp.int32),
        jax.ShapeDtypeStruct((1, LANES), jnp.int32),
    ),
)


_RPW = NB // NW
_ZPW = NB // 16
_PPS = 2 * N // 16


def _dispatch_body(pos2_hbm, x_hbm, ur_hbm, xs_hbm, zbuf, pslice, vbuf,
                   myids, uv, rb0, rb1, rb2, ts_sh, semg, sems):
    s = lax.axis_index("s")
    c = lax.axis_index("c")

    def zb(i, carry):
        zbuf[pl.ds(i * 16, 16)] = jnp.zeros((16,), jnp.int32)
        return carry
    lax.fori_loop(0, _ZPW // 16, zb, 0)
    pltpu.sync_copy(zbuf, ts_sh.at[pl.ds(s * _ZPW, _ZPW)])

    pltpu.sync_copy(pos2_hbm.at[pl.ds(s * 2, 2)], pslice)
    for j in range(2):
        def vb(i, carry, j=j):
            vbuf[j, pl.ds(i * 16, 16)] = (
                (s * _PPS + j * 128 + i * 16 + lax.iota(jnp.int32, 16))
                & (N - 1))
            return carry
        lax.fori_loop(0, 8, vb, 0)
    plsc.subcore_barrier()

    for j in range(2):
        pltpu.sync_copy(vbuf.at[j], ts_sh.at[pslice.at[j]], add=True)
    plsc.subcore_barrier()

    wid = s * 2 + c
    base = wid * _RPW
    pltpu.sync_copy(ur_hbm, uv)
    nrows = uv[...][0]

    @pl.when(base < nrows)
    def _phase2():
        pltpu.sync_copy(ts_sh.at[pl.ds(base, _RPW)], myids)
        cw = 40
        nch = _RPW // cw
        rbs = (rb0, rb1, rb2)
        g = [None, None, None]
        st = [None, None, None]
        for ch in range(min(3, nch)):
            g[ch] = pltpu.async_copy(
                x_hbm.at[myids.at[pl.ds(ch * cw, cw)]], rbs[ch], semg)
        for ch in range(nch):
            b = ch % 3
            g[b].wait()
            st[b] = pltpu.async_copy(
                rbs[b], xs_hbm.at[pl.ds(base + ch * cw, cw)], sems)
            if ch + 3 < nch:
                st[b].wait()
                g[b] = pltpu.async_copy(
                    x_hbm.at[myids.at[pl.ds((ch + 3) * cw, cw)]],
                    rbs[b], semg)
        for ch in range(max(0, nch - 3), nch):
            if st[ch % 3] is not None:
                st[ch % 3].wait()
                st[ch % 3] = None


def _ffn_body(te_ref, xs_ref, w1_ref, w2_ref, y_ref):
    @pl.when(pl.program_id(0) < te_ref[NT])
    def _():
        h = jnp.dot(xs_ref[...], w1_ref[0],
                    preferred_element_type=jnp.float32)
        h = 0.5 * h * (1.0 + lax.erf(h * 0.7071067811865476))
        y_ref[...] = jnp.dot(h, w2_ref[0],
                             preferred_element_type=jnp.float32)


_ffn = pl.pallas_call(
    _ffn_body,
    grid_spec=pltpu.PrefetchScalarGridSpec(
        num_scalar_prefetch=1,
        grid=(NT,),
        in_specs=[
            pl.BlockSpec((TM, D), lambda t, te: (t, 0)),
            pl.BlockSpec((1, D, F), lambda t, te: (te[t], 0, 0)),
            pl.BlockSpec((1, F, D), lambda t, te: (te[t], 0, 0)),
        ],
        out_specs=pl.BlockSpec((TM, D), lambda t, te: (t, 0)),
    ),
    out_shape=jax.ShapeDtypeStruct((NB, D), jnp.float32),
    compiler_params=pltpu.CompilerParams(
        dimension_semantics=("arbitrary",)),
)


_TPW = N // NW


_CCH = _TPW // 16


def _combine_body(y_hbm, pos_hbm, tpa_hbm, tpb_hbm, out_hbm, iaall,
                  iball, taall, tball, ba0, bb0, ba1, bb1, ob0, ob1,
                  sema, semb, semo):
    s = lax.axis_index("s")
    c = lax.axis_index("c")
    wid = s * 2 + c
    base = wid * _TPW

    pltpu.sync_copy(pos_hbm.at[pl.ds(base, _TPW)], iaall)
    pltpu.sync_copy(pos_hbm.at[pl.ds(N + base, _TPW)], iball)
    pltpu.sync_copy(tpa_hbm.at[pl.ds(base, _TPW)], taall)
    pltpu.sync_copy(tpb_hbm.at[pl.ds(base, _TPW)], tball)

    ca = pltpu.async_copy(y_hbm.at[iaall.at[pl.ds(0, 16)]], ba0, sema)
    cb = pltpu.async_copy(y_hbm.at[iball.at[pl.ds(0, 16)]], bb0, semb)
    so = [None, None]
    for ch in range(_CCH):
        par = ch % 2
        ba, bb = (ba0, bb0) if par == 0 else (ba1, bb1)
        nba, nbb = (ba1, bb1) if par == 0 else (ba0, bb0)
        obuf = ob0 if par == 0 else ob1
        t0 = base + ch * 16
        ca.wait()
        cb.wait()
        if ch + 1 < _CCH:
            nsl = pl.ds((ch + 1) * 16, 16)
            ca = pltpu.async_copy(y_hbm.at[iaall.at[nsl]], nba, sema)
            cb = pltpu.async_copy(y_hbm.at[iball.at[nsl]], nbb, semb)
        if so[par] is not None:
            so[par].wait()
        tav = taall[pl.ds(ch * 16, 16)]
        tbv = tball[pl.ds(ch * 16, 16)]
        for i in range(16):
            a = tav[i]
            b = tbv[i]

            def row_body(j, carry, i=i, a=a, b=b, ba=ba, bb=bb,
                         obuf=obuf):
                for jj in range(4):
                    sl = pl.ds(j * 64 + jj * 16, 16)
                    obuf[i, sl] = ba[i, sl] * a + bb[i, sl] * b
                return carry
            lax.fori_loop(0, D // 64, row_body, 0)
        so[par] = pltpu.async_copy(obuf, out_hbm.at[pl.ds(t0, 16)], semo)
    for p in range(2):
        if so[p] is not None:
            so[p].wait()


@functools.cache
def _sc_kernels():
    mesh = plsc.VectorSubcoreMesh(core_axis_name="c", subcore_axis_name="s")
    dispatch = pl.kernel(
        _dispatch_body,
        mesh=mesh,
        out_type=jax.ShapeDtypeStruct((NB, D), jnp.float32),
        scratch_types=[
            pltpu.VMEM((_ZPW,), jnp.int32),
            pltpu.VMEM((2, 128), jnp.int32),
            pltpu.VMEM((2, 128), jnp.int32),
            pltpu.VMEM((_RPW,), jnp.int32),
            pltpu.VMEM((16,), jnp.int32),
            pltpu.VMEM((40, D), jnp.float32),
            pltpu.VMEM((40, D), jnp.float32),
            pltpu.VMEM((40, D), jnp.float32),
            pltpu.VMEM_SHARED((NB,), jnp.int32),
            pltpu.SemaphoreType.DMA,
            pltpu.SemaphoreType.DMA,
        ],
        compiler_params=pltpu.CompilerParams(needs_layout_passes=False),
    )
    combine = pl.kernel(
        _combine_body,
        mesh=mesh,
        out_type=jax.ShapeDtypeStruct((N, D), jnp.float32),
        scratch_types=[
            pltpu.VMEM((_TPW,), jnp.int32),
            pltpu.VMEM((_TPW,), jnp.int32),
            pltpu.VMEM((_TPW,), jnp.float32),
            pltpu.VMEM((_TPW,), jnp.float32),
            pltpu.VMEM((16, D), jnp.float32),
            pltpu.VMEM((16, D), jnp.float32),
            pltpu.VMEM((16, D), jnp.float32),
            pltpu.VMEM((16, D), jnp.float32),
            pltpu.VMEM((16, D), jnp.float32),
            pltpu.VMEM((16, D), jnp.float32),
            pltpu.SemaphoreType.DMA,
            pltpu.SemaphoreType.DMA,
            pltpu.SemaphoreType.DMA,
        ],
    )
    return dispatch, combine


def kernel(x, gate_w, w1, w2):
    Bb, Tt, Dd = x.shape
    x_flat = x.reshape(N, D)
    gwp = jnp.pad(gate_w, ((0, 0), (0, LANES - E)))
    probs, ti, tp, aux, pos, te2d, ur = _router(x_flat, gwp)
    te_ext = te2d[:NT + 1, 0]
    used_rows = ur[0, :16]

    dispatch, combine = _sc_kernels()
    pos_flat = pos.reshape(2 * N)
    xs = dispatch(pos_flat.reshape(32, 128), x_flat, used_rows)
    y = _ffn(te_ext, xs, w1, w2)
    out = combine(y, pos_flat, tp[:, 0], tp[:, 1])

    return (out.reshape(Bb, Tt, Dd), aux.reshape(()),
            probs.reshape(Bb, Tt, E), ti.reshape(Bb, Tt, K),
            tp.reshape(Bb, Tt, K))
